# baseline (speedup 1.0000x reference)
import functools

import jax
import jax.numpy as jnp
from jax import lax
from jax.experimental import pallas as pl
from jax.experimental.pallas import tpu as pltpu
from jax.experimental.pallas import tpu_sc as plsc

_N = 10000
_D = 256
_NP = 10240
_TRASH = 10000
_NSC = 2
_NSUB = 16
_CHUNK = _NP // _NSUB
_B = 128
_AGG_BLOCKS = 80
_PAIRS = _AGG_BLOCKS // 2

_mesh = plsc.VectorSubcoreMesh(core_axis_name="c", subcore_axis_name="s")


_DEG_SPLIT = _PAIRS


@functools.partial(
    pl.kernel,
    mesh=_mesh,
    out_type=jax.ShapeDtypeStruct((_NSC, _NP, 128), jnp.float32),
    scratch_types=[
        pltpu.VMEM_SHARED((_NP, 128), jnp.float32),
        pltpu.VMEM((_B,), jnp.int32),
        pltpu.VMEM((_B, 128), jnp.float32),
    ],
)
def _deg_call(dst_hbm, out_hbm, acc_sh, dstidx_v, buf_v):
    c = lax.axis_index("c")
    s = lax.axis_index("s")

    def fill(val):
        def row(i, carry):
            def lane(j, carry2):
                buf_v[i, pl.ds(j * 16, 16)] = val
                return carry2
            return lax.fori_loop(0, 8, lane, carry)
        lax.fori_loop(0, _B, row, 0)

    fill(jnp.zeros((16,), jnp.float32))

    def ini(k, carry):
        pltpu.sync_copy(buf_v, acc_sh.at[pl.ds(s * _CHUNK + k * _B, _B)])
        return carry

    lax.fori_loop(0, _CHUNK // _B, ini, 0)
    fill(jnp.ones((16,), jnp.float32))
    plsc.subcore_barrier()

    def blk(j, carry):
        pltpu.sync_copy(dst_hbm.at[s, j], dstidx_v)
        pltpu.sync_copy(buf_v, acc_sh.at[dstidx_v], add=True)
        return carry

    lax.fori_loop(c * _DEG_SPLIT,
                  _DEG_SPLIT + c * (_AGG_BLOCKS - _DEG_SPLIT), blk, 0)
    plsc.subcore_barrier()

    def fin(k, carry):
        off = s * _CHUNK + k * _B
        pltpu.sync_copy(acc_sh.at[pl.ds(off, _B)], buf_v)
        pltpu.sync_copy(buf_v, out_hbm.at[c, pl.ds(off, _B)])
        return carry

    lax.fori_loop(0, _CHUNK // _B, fin, 0)


@functools.partial(
    pl.kernel,
    mesh=_mesh,
    out_type=jax.ShapeDtypeStruct((_NSC, _NP, 128), jnp.float32),
    scratch_types=[
        pltpu.VMEM_SHARED((_NP, 128), jnp.float32),
        pltpu.VMEM((_AGG_BLOCKS * _B,), jnp.int32),
        pltpu.VMEM((_B,), jnp.int32),
        pltpu.VMEM((_B, 128), jnp.float32),
        pltpu.SemaphoreType.DMA,
    ],
)
def _agg_call(g_hbm, src_hbm, dst_hbm, out_hbm, acc_sh, srcall_v, dstidx_v,
              rows_v, sem):
    c = lax.axis_index("c")
    s = lax.axis_index("s")
    base = c * _NP

    pltpu.sync_copy(src_hbm.at[s], srcall_v)

    def addbase(i, carry):
        srcall_v[pl.ds(i * 16, 16)] = srcall_v[pl.ds(i * 16, 16)] + base
        return carry

    lax.fori_loop(0, _AGG_BLOCKS * _B // 16, addbase, 0)

    def ini(k, carry):
        off = s * _CHUNK + k * _B
        pltpu.sync_copy(g_hbm.at[pl.ds(base + off, _B)], rows_v)
        pltpu.sync_copy(rows_v, acc_sh.at[pl.ds(off, _B)])
        return carry

    lax.fori_loop(0, _CHUNK // _B, ini, 0)
    plsc.subcore_barrier()

    def blk(j, carry):
        pltpu.sync_copy(dst_hbm.at[s, j], dstidx_v)
        pltpu.async_copy(
            g_hbm.at[srcall_v.at[pl.ds(j * _B, _B)]], rows_v, sem).wait()
        pltpu.sync_copy(rows_v, acc_sh.at[dstidx_v], add=True)
        return carry

    lax.fori_loop(0, _AGG_BLOCKS, blk, 0)
    plsc.subcore_barrier()

    def fin(k, carry):
        off = s * _CHUNK + k * _B
        pltpu.sync_copy(acc_sh.at[pl.ds(off, _B)], rows_v)
        pltpu.sync_copy(rows_v, out_hbm.at[c, pl.ds(off, _B)])
        return carry

    lax.fori_loop(0, _CHUNK // _B, fin, 0)


_BLK = 2048


def _tc1a_body(x_ref, w1_ref, h_ref):
    h_ref[...] = jnp.dot(x_ref[...], w1_ref[...],
                         preferred_element_type=jnp.float32)


def _tc1b_body(h_ref, dg0_ref, dg1_ref, g_ref, dinv_ref):
    deg = dg0_ref[0, :, :1] + dg1_ref[0, :, :1] + 1.0
    dinv = lax.rsqrt(deg)
    g = h_ref[...] * dinv
    g_ref[0] = g[:, :128]
    g_ref[1] = g[:, 128:]
    dinv_ref[...] = dinv


def _tc2_body(a0_ref, a1_ref, dinv_ref, b1_ref, w2_ref, g_ref):
    dinv = dinv_ref[...]
    hin = jnp.concatenate([a0_ref[0], a1_ref[0]], axis=1)
    o1 = jnp.maximum(hin * dinv + b1_ref[...], 0.0)
    h2 = jnp.dot(o1, w2_ref[...], preferred_element_type=jnp.float32)
    g2 = h2 * dinv
    g_ref[0] = g2[:, :128]
    g_ref[1] = g2[:, 128:]


def _tc3_body(a0_ref, a1_ref, dinv_ref, b2_ref, out_ref):
    hin = jnp.concatenate([a0_ref[0], a1_ref[0]], axis=1)
    out_ref[...] = hin * dinv_ref[...] + b2_ref[...]


_tc1a = pl.pallas_call(
    _tc1a_body,
    grid=(_NP // _BLK,),
    in_specs=[
        pl.BlockSpec((_BLK, _D), lambda r: (r, 0)),
        pl.BlockSpec((_D, _D), lambda r: (0, 0)),
    ],
    out_specs=pl.BlockSpec((_BLK, _D), lambda r: (r, 0)),
    out_shape=jax.ShapeDtypeStruct((_NP, _D), jnp.float32),
)

_tc1b = pl.pallas_call(
    _tc1b_body,
    grid=(_NP // _BLK,),
    in_specs=[
        pl.BlockSpec((_BLK, _D), lambda r: (r, 0)),
        pl.BlockSpec((1, _BLK, 128), lambda r: (0, r, 0)),
        pl.BlockSpec((1, _BLK, 128), lambda r: (1, r, 0)),
    ],
    out_specs=[
        pl.BlockSpec((2, _BLK, 128), lambda r: (0, r, 0)),
        pl.BlockSpec((_BLK, 1), lambda r: (r, 0)),
    ],
    out_shape=[
        jax.ShapeDtypeStruct((_NSC, _NP, 128), jnp.float32),
        jax.ShapeDtypeStruct((_NP, 1), jnp.float32),
    ],
)

_tc2 = pl.pallas_call(
    _tc2_body,
    grid=(_NP // _BLK,),
    in_specs=[
        pl.BlockSpec((1, _BLK, 128), lambda r: (0, r, 0)),
        pl.BlockSpec((1, _BLK, 128), lambda r: (1, r, 0)),
        pl.BlockSpec((_BLK, 1), lambda r: (r, 0)),
        pl.BlockSpec((1, _D), lambda r: (0, 0)),
        pl.BlockSpec((_D, _D), lambda r: (0, 0)),
    ],
    out_specs=pl.BlockSpec((2, _BLK, 128), lambda r: (0, r, 0)),
    out_shape=jax.ShapeDtypeStruct((_NSC, _NP, 128), jnp.float32),
)

_tc3 = pl.pallas_call(
    _tc3_body,
    grid=(_NP // _BLK,),
    in_specs=[
        pl.BlockSpec((1, _BLK, 128), lambda r: (0, r, 0)),
        pl.BlockSpec((1, _BLK, 128), lambda r: (1, r, 0)),
        pl.BlockSpec((_BLK, 1), lambda r: (r, 0)),
        pl.BlockSpec((1, _D), lambda r: (0, 0)),
    ],
    out_specs=pl.BlockSpec((_BLK, _D), lambda r: (r, 0)),
    out_shape=jax.ShapeDtypeStruct((_N, _D), jnp.float32),
)


def kernel(x, adj, W1, b1, W2, b2):
    e = adj.shape[1]
    src = adj[0]
    dst = adj[1]

    ep = _NSUB * _AGG_BLOCKS * _B
    src_t = jnp.concatenate(
        [src, jnp.zeros((ep - e,), jnp.int32)]).reshape(
            _NSUB, _AGG_BLOCKS * _B)
    dst_t = jnp.concatenate(
        [dst, jnp.full((ep - e,), _TRASH, jnp.int32)]).reshape(
            _NSUB, _AGG_BLOCKS, _B)

    deg2 = _deg_call(dst_t)
    h1 = _tc1a(x, W1)
    g1, dinv = _tc1b(h1, deg2, deg2)
    agg1 = _agg_call(g1.reshape(_NSC * _NP, 128), src_t, dst_t)
    g2 = _tc2(agg1, agg1, dinv, b1.reshape(1, _D), W2)
    agg2 = _agg_call(g2.reshape(_NSC * _NP, 128), src_t, dst_t)
    return _tc3(agg2, agg2, dinv, b2.reshape(1, _D))


# speedup vs baseline: 1.3341x; 1.3341x over previous
import functools

import jax
import jax.numpy as jnp
from jax import lax
from jax.experimental import pallas as pl
from jax.experimental.pallas import tpu as pltpu
from jax.experimental.pallas import tpu_sc as plsc

_N = 10000
_D = 256
_NP = 10240
_TRASH = 10000
_NSC = 2
_NSUB = 16
_CHUNK = _NP // _NSUB
_B = 128
_AGG_BLOCKS = 79

_mesh = plsc.VectorSubcoreMesh(core_axis_name="c", subcore_axis_name="s")


_DEG_SPLIT = 40


@functools.partial(
    pl.kernel,
    mesh=_mesh,
    out_type=jax.ShapeDtypeStruct((_NSC, _NP, 128), jnp.float32),
    scratch_types=[
        pltpu.VMEM_SHARED((_NP, 128), jnp.float32),
        pltpu.VMEM((_B,), jnp.int32),
        pltpu.VMEM((_B, 128), jnp.float32),
    ],
)
def _deg_call(dst_hbm, out_hbm, acc_sh, dstidx_v, buf_v):
    c = lax.axis_index("c")
    s = lax.axis_index("s")

    def fill(val):
        def row(i, carry):
            def lane(j, carry2):
                buf_v[i, pl.ds(j * 16, 16)] = val
                return carry2
            return lax.fori_loop(0, 8, lane, carry)
        lax.fori_loop(0, _B, row, 0)

    fill(jnp.zeros((16,), jnp.float32))

    def ini(k, carry):
        pltpu.sync_copy(buf_v, acc_sh.at[pl.ds(s * _CHUNK + k * _B, _B)])
        return carry

    lax.fori_loop(0, _CHUNK // _B, ini, 0)
    fill(jnp.ones((16,), jnp.float32))
    plsc.subcore_barrier()

    def blk(j, carry):
        pltpu.sync_copy(dst_hbm.at[s, j], dstidx_v)
        pltpu.sync_copy(buf_v, acc_sh.at[dstidx_v], add=True)
        return carry

    lax.fori_loop(c * _DEG_SPLIT,
                  _DEG_SPLIT + c * (_AGG_BLOCKS - _DEG_SPLIT), blk, 0)
    plsc.subcore_barrier()

    def fin(k, carry):
        off = s * _CHUNK + k * _B
        pltpu.sync_copy(acc_sh.at[pl.ds(off, _B)], buf_v)
        pltpu.sync_copy(buf_v, out_hbm.at[c, pl.ds(off, _B)])
        return carry

    lax.fori_loop(0, _CHUNK // _B, fin, 0)


@functools.partial(
    pl.kernel,
    mesh=_mesh,
    out_type=jax.ShapeDtypeStruct((_NSC, _NP, 128), jnp.float32),
    scratch_types=[
        pltpu.VMEM_SHARED((_NP, 128), jnp.float32),
        pltpu.VMEM((_AGG_BLOCKS * _B,), jnp.int32),
        pltpu.VMEM((_B,), jnp.int32),
        pltpu.VMEM((_B, 128), jnp.float32),
        pltpu.SemaphoreType.DMA,
    ],
)
def _agg_call(g_hbm, src_hbm, dst_hbm, out_hbm, acc_sh, srcall_v, dstidx_v,
              rows_v, sem):
    c = lax.axis_index("c")
    s = lax.axis_index("s")
    base = c * _NP

    pltpu.sync_copy(src_hbm.at[s], srcall_v)

    def addbase(i, carry):
        srcall_v[pl.ds(i * 16, 16)] = srcall_v[pl.ds(i * 16, 16)] + base
        return carry

    lax.fori_loop(0, _AGG_BLOCKS * _B // 16, addbase, 0)

    def ini(k, carry):
        off = s * _CHUNK + k * _B
        pltpu.sync_copy(g_hbm.at[pl.ds(base + off, _B)], rows_v)
        pltpu.sync_copy(rows_v, acc_sh.at[pl.ds(off, _B)])
        return carry

    lax.fori_loop(0, _CHUNK // _B, ini, 0)
    plsc.subcore_barrier()

    def blk(j, carry):
        pltpu.sync_copy(dst_hbm.at[s, j], dstidx_v)
        pltpu.async_copy(
            g_hbm.at[srcall_v.at[pl.ds(j * _B, _B)]], rows_v, sem).wait()
        pltpu.sync_copy(rows_v, acc_sh.at[dstidx_v], add=True)
        return carry

    lax.fori_loop(0, _AGG_BLOCKS, blk, 0)
    plsc.subcore_barrier()

    def fin(k, carry):
        off = s * _CHUNK + k * _B
        pltpu.sync_copy(acc_sh.at[pl.ds(off, _B)], rows_v)
        pltpu.sync_copy(rows_v, out_hbm.at[c, pl.ds(off, _B)])
        return carry

    lax.fori_loop(0, _CHUNK // _B, fin, 0)


_BLK = 2048


def _tc1_body(x_ref, w1_ref, dg0_ref, dg1_ref, g_ref, dinv_ref):
    deg = dg0_ref[0, :, :1] + dg1_ref[0, :, :1] + 1.0
    dinv = lax.rsqrt(deg)
    h = jnp.dot(x_ref[...], w1_ref[...], preferred_element_type=jnp.float32)
    g = h * dinv
    g_ref[0] = g[:, :128]
    g_ref[1] = g[:, 128:]
    dinv_ref[...] = dinv


def _tc2_body(a0_ref, a1_ref, dinv_ref, b1_ref, w2_ref, g_ref):
    dinv = dinv_ref[...]
    hin = jnp.concatenate([a0_ref[0], a1_ref[0]], axis=1)
    o1 = jnp.maximum(hin * dinv + b1_ref[...], 0.0)
    h2 = jnp.dot(o1, w2_ref[...], preferred_element_type=jnp.float32)
    g2 = h2 * dinv
    g_ref[0] = g2[:, :128]
    g_ref[1] = g2[:, 128:]


def _tc3_body(a0_ref, a1_ref, dinv_ref, b2_ref, out_ref):
    hin = jnp.concatenate([a0_ref[0], a1_ref[0]], axis=1)
    out_ref[...] = hin * dinv_ref[...] + b2_ref[...]


_tc1 = pl.pallas_call(
    _tc1_body,
    grid=(_NP // _BLK,),
    in_specs=[
        pl.BlockSpec((_BLK, _D), lambda r: (r, 0)),
        pl.BlockSpec((_D, _D), lambda r: (0, 0)),
        pl.BlockSpec((1, _BLK, 128), lambda r: (0, r, 0)),
        pl.BlockSpec((1, _BLK, 128), lambda r: (1, r, 0)),
    ],
    out_specs=[
        pl.BlockSpec((2, _BLK, 128), lambda r: (0, r, 0)),
        pl.BlockSpec((_BLK, 1), lambda r: (r, 0)),
    ],
    out_shape=[
        jax.ShapeDtypeStruct((_NSC, _NP, 128), jnp.float32),
        jax.ShapeDtypeStruct((_NP, 1), jnp.float32),
    ],
)

_tc2 = pl.pallas_call(
    _tc2_body,
    grid=(_NP // _BLK,),
    in_specs=[
        pl.BlockSpec((1, _BLK, 128), lambda r: (0, r, 0)),
        pl.BlockSpec((1, _BLK, 128), lambda r: (1, r, 0)),
        pl.BlockSpec((_BLK, 1), lambda r: (r, 0)),
        pl.BlockSpec((1, _D), lambda r: (0, 0)),
        pl.BlockSpec((_D, _D), lambda r: (0, 0)),
    ],
    out_specs=pl.BlockSpec((2, _BLK, 128), lambda r: (0, r, 0)),
    out_shape=jax.ShapeDtypeStruct((_NSC, _NP, 128), jnp.float32),
)

_tc3 = pl.pallas_call(
    _tc3_body,
    grid=(_NP // _BLK,),
    in_specs=[
        pl.BlockSpec((1, _BLK, 128), lambda r: (0, r, 0)),
        pl.BlockSpec((1, _BLK, 128), lambda r: (1, r, 0)),
        pl.BlockSpec((_BLK, 1), lambda r: (r, 0)),
        pl.BlockSpec((1, _D), lambda r: (0, 0)),
    ],
    out_specs=pl.BlockSpec((_BLK, _D), lambda r: (r, 0)),
    out_shape=jax.ShapeDtypeStruct((_N, _D), jnp.float32),
)


def kernel(x, adj, W1, b1, W2, b2):
    e = adj.shape[1]
    src = adj[0]
    dst = adj[1]

    ep = _NSUB * _AGG_BLOCKS * _B
    src_t = jnp.concatenate(
        [src, jnp.zeros((ep - e,), jnp.int32)]).reshape(
            _NSUB, _AGG_BLOCKS * _B)
    dst_t = jnp.concatenate(
        [dst, jnp.full((ep - e,), _TRASH, jnp.int32)]).reshape(
            _NSUB, _AGG_BLOCKS, _B)

    deg2 = _deg_call(dst_t)
    g1, dinv = _tc1(x, W1, deg2, deg2)
    agg1 = _agg_call(g1.reshape(_NSC * _NP, 128), src_t, dst_t)
    g2 = _tc2(agg1, agg1, dinv, b1.reshape(1, _D), W2)
    agg2 = _agg_call(g2.reshape(_NSC * _NP, 128), src_t, dst_t)
    return _tc3(agg2, agg2, dinv, b2.reshape(1, _D))
